# baseline (speedup 1.0000x reference)
import functools

import jax
import jax.numpy as jnp
from jax import lax
from jax.experimental import pallas as pl
from jax.experimental.pallas import tpu as pltpu
from jax.experimental.pallas import tpu_sc as plsc

_R = 2.0
_T = 1.0

_NC = 2
_NS = 16
_NW = _NC * _NS
_L = 16
_B = 64
_X0 = 80
_X1 = 80


def _worker(n_chunks, base, emb_hbm, src_hbm, dst_hbm, out_hbm,
            idxs_v, idxd_v, srcbuf, dstbuf, sums_v, outw_v, sem_s, sem_d):
    ew = n_chunks * _B
    d32 = srcbuf.shape[1]
    nj = d32 // _L

    pltpu.sync_copy(src_hbm.at[pl.ds(base, ew)], idxs_v.at[pl.ds(0, ew)])
    pltpu.sync_copy(dst_hbm.at[pl.ds(base, ew)], idxd_v.at[pl.ds(0, ew)])

    lane15 = lax.iota(jnp.int32, _L) * _L + (_L - 1)

    def chunk_body(c, carry):
        cs = c * _B
        cp1 = pltpu.async_copy(emb_hbm.at[idxs_v.at[pl.ds(cs, _B)]],
                               srcbuf, sem_s)
        cp2 = pltpu.async_copy(emb_hbm.at[idxd_v.at[pl.ds(cs, _B)]],
                               dstbuf, sem_d)
        cp1.wait()
        cp2.wait()

        himask = jnp.full((_L,), -65536, dtype=jnp.int32)

        def edge_body(e, carry2):
            acc = jnp.zeros((_L,), jnp.float32)
            for j in range(nj):
                a32 = srcbuf[e, pl.ds(j * _L, _L)]
                b32 = dstbuf[e, pl.ds(j * _L, _L)]
                ah = plsc.bitcast(a32 & himask, jnp.float32)
                al = plsc.bitcast(a32 << 16, jnp.float32)
                bh = plsc.bitcast(b32 & himask, jnp.float32)
                bl = plsc.bitcast(b32 << 16, jnp.float32)
                dh = ah - bh
                dl = al - bl
                acc = acc + dh * dh + dl * dl
            sums_v[pl.ds(e * _L, _L)] = plsc.cumsum(acc)
            return carry2

        lax.fori_loop(0, _B, edge_body, 0, unroll=2)

        def group_body(g, carry2):
            idx = g * (_L * _L) + lane15
            s = plsc.load_gather(sums_v, [idx])
            probs = 1.0 / (jnp.exp((s - _R) * (1.0 / _T)) + 1.0)
            outw_v[pl.ds(cs + g * _L, _L)] = probs
            return carry2

        lax.fori_loop(0, _B // _L, group_body, 0)
        return carry

    lax.fori_loop(0, n_chunks, chunk_body, 0)
    pltpu.sync_copy(outw_v.at[pl.ds(0, ew)], out_hbm.at[pl.ds(base, ew)])


def _sc_body(emb_hbm, src_hbm, dst_hbm, out_hbm,
             idxs_v, idxd_v, srcbuf, dstbuf, sums_v, outw_v, sem_s, sem_d):
    c = lax.axis_index("c")
    s = lax.axis_index("s")
    args = (emb_hbm, src_hbm, dst_hbm, out_hbm,
            idxs_v, idxd_v, srcbuf, dstbuf, sums_v, outw_v, sem_s, sem_d)

    if _X0 == _X1:
        wid = s * _NC + c
        _worker(_X0, wid * (_X0 * _B), *args)
    else:
        @pl.when(c == 0)
        def _():
            _worker(_X0, s * (_X0 * _B), *args)

        @pl.when(c == 1)
        def _():
            _worker(_X1, _NS * (_X0 * _B) + s * (_X1 * _B), *args)


@jax.jit
def kernel(shower, embeddings, edge_index):
    del shower
    e_total = edge_index.shape[1]
    d = embeddings.shape[1]
    e_pad = _NS * (_X0 + _X1) * _B
    assert e_pad >= e_total
    ew_max = _X0 * _B

    emb16 = embeddings.astype(jnp.bfloat16)
    emb32 = lax.bitcast_convert_type(
        emb16.reshape(emb16.shape[0], d // 2, 2), jnp.int32)
    src = edge_index[0]
    dst = edge_index[1]
    pad = e_pad - e_total
    if pad:
        zpad = jnp.zeros((pad,), jnp.int32)
        src = jnp.concatenate([src, zpad])
        dst = jnp.concatenate([dst, zpad])

    mesh = plsc.VectorSubcoreMesh(core_axis_name="c", subcore_axis_name="s")
    fn = pl.kernel(
        _sc_body,
        out_type=jax.ShapeDtypeStruct((e_pad,), jnp.float32),
        mesh=mesh,
        compiler_params=pltpu.CompilerParams(needs_layout_passes=False),
        scratch_types=[
            pltpu.VMEM((ew_max,), jnp.int32),
            pltpu.VMEM((ew_max,), jnp.int32),
            pltpu.VMEM((_B, d // 2), jnp.int32),
            pltpu.VMEM((_B, d // 2), jnp.int32),
            pltpu.VMEM((_B * _L,), jnp.float32),
            pltpu.VMEM((ew_max,), jnp.float32),
            pltpu.SemaphoreType.DMA,
            pltpu.SemaphoreType.DMA,
        ],
    )
    out = fn(emb32, src, dst)
    return out[:e_total]


# speedup vs baseline: 1.0350x; 1.0350x over previous
import functools

import jax
import jax.numpy as jnp
from jax import lax
from jax.experimental import pallas as pl
from jax.experimental.pallas import tpu as pltpu
from jax.experimental.pallas import tpu_sc as plsc

_R = 2.0
_T = 1.0

_NC = 2
_NS = 16
_NW = _NC * _NS
_L = 16
_B = 64
_X0 = 94
_X1 = 66


def _worker(n_chunks, base, emb_hbm, src_hbm, dst_hbm, out_hbm,
            idxs_v, idxd_v, srcbuf, dstbuf, sums_v, outw_v, sem_s, sem_d):
    ew = n_chunks * _B
    nj = srcbuf.shape[1] // _L

    pltpu.sync_copy(src_hbm.at[pl.ds(base, ew)], idxs_v.at[pl.ds(0, ew)])
    pltpu.sync_copy(dst_hbm.at[pl.ds(base, ew)], idxd_v.at[pl.ds(0, ew)])

    lane15 = lax.iota(jnp.int32, _L) * _L + (_L - 1)

    def chunk_body(c, carry):
        cs = c * _B
        cp1 = pltpu.async_copy(emb_hbm.at[idxs_v.at[pl.ds(cs, _B)]],
                               srcbuf, sem_s)
        cp2 = pltpu.async_copy(emb_hbm.at[idxd_v.at[pl.ds(cs, _B)]],
                               dstbuf, sem_d)
        cp1.wait()
        cp2.wait()

        def edge_body(e, carry2):
            acc = jnp.zeros((_L,), jnp.float32)
            for j in range(nj):
                a = srcbuf[e, pl.ds(j * _L, _L)]
                bb = dstbuf[e, pl.ds(j * _L, _L)]
                diff = a - bb
                acc = acc + diff * diff
            sums_v[pl.ds(e * _L, _L)] = plsc.cumsum(acc)
            return carry2

        lax.fori_loop(0, _B, edge_body, 0, unroll=2)

        def group_body(g, carry2):
            idx = g * (_L * _L) + lane15
            s = plsc.load_gather(sums_v, [idx])
            probs = 1.0 / (jnp.exp((s - _R) * (1.0 / _T)) + 1.0)
            outw_v[pl.ds(cs + g * _L, _L)] = probs
            return carry2

        lax.fori_loop(0, _B // _L, group_body, 0)
        return carry

    lax.fori_loop(0, n_chunks, chunk_body, 0)
    pltpu.sync_copy(outw_v.at[pl.ds(0, ew)], out_hbm.at[pl.ds(base, ew)])


def _sc_body(emb_hbm, src_hbm, dst_hbm, out_hbm,
             idxs_v, idxd_v, srcbuf, dstbuf, sums_v, outw_v, sem_s, sem_d):
    c = lax.axis_index("c")
    s = lax.axis_index("s")
    args = (emb_hbm, src_hbm, dst_hbm, out_hbm,
            idxs_v, idxd_v, srcbuf, dstbuf, sums_v, outw_v, sem_s, sem_d)

    if _X0 == _X1:
        wid = s * _NC + c
        _worker(_X0, wid * (_X0 * _B), *args)
    else:
        @pl.when(c == 0)
        def _():
            _worker(_X0, s * (_X0 * _B), *args)

        @pl.when(c == 1)
        def _():
            _worker(_X1, _NS * (_X0 * _B) + s * (_X1 * _B), *args)


@jax.jit
def kernel(shower, embeddings, edge_index):
    del shower
    e_total = edge_index.shape[1]
    d = embeddings.shape[1]
    e_pad = _NS * (_X0 + _X1) * _B
    assert e_pad >= e_total
    ew_max = _X0 * _B

    src = edge_index[0]
    dst = edge_index[1]
    pad = e_pad - e_total
    if pad:
        zpad = jnp.zeros((pad,), jnp.int32)
        src = jnp.concatenate([src, zpad])
        dst = jnp.concatenate([dst, zpad])

    mesh = plsc.VectorSubcoreMesh(core_axis_name="c", subcore_axis_name="s")
    fn = pl.kernel(
        _sc_body,
        out_type=jax.ShapeDtypeStruct((e_pad,), jnp.float32),
        mesh=mesh,
        compiler_params=pltpu.CompilerParams(needs_layout_passes=False),
        scratch_types=[
            pltpu.VMEM((ew_max,), jnp.int32),
            pltpu.VMEM((ew_max,), jnp.int32),
            pltpu.VMEM((_B, d), jnp.float32),
            pltpu.VMEM((_B, d), jnp.float32),
            pltpu.VMEM((_B * _L,), jnp.float32),
            pltpu.VMEM((ew_max,), jnp.float32),
            pltpu.SemaphoreType.DMA,
            pltpu.SemaphoreType.DMA,
        ],
    )
    out = fn(embeddings, src, dst)
    return out[:e_total]


# speedup vs baseline: 1.2048x; 1.1641x over previous
import functools

import jax
import jax.numpy as jnp
from jax import lax
from jax.experimental import pallas as pl
from jax.experimental.pallas import tpu as pltpu
from jax.experimental.pallas import tpu_sc as plsc

_R = 2.0
_T = 1.0

_NC = 2
_NS = 16
_NW = _NC * _NS
_L = 16
_B = 32
_NBUF = 4


def _sc_body(n_chunks, emb_hbm, src_hbm, dst_hbm, out_hbm,
             idxs_v, idxd_v, srcbufs, dstbufs, sums_v, outw_v, *sems):
    ew = n_chunks * _B
    nj = srcbufs.shape[2] // _L
    sems_s = sems[:_NBUF]
    sems_d = sems[_NBUF:]
    wid = lax.axis_index("s") * _NC + lax.axis_index("c")
    base = wid * ew

    pltpu.sync_copy(src_hbm.at[pl.ds(base, ew)], idxs_v)
    pltpu.sync_copy(dst_hbm.at[pl.ds(base, ew)], idxd_v)

    lane15 = lax.iota(jnp.int32, _L) * _L + (_L - 1)

    def start(chunk, b):
        cs = chunk * _B
        pltpu.async_copy(emb_hbm.at[idxs_v.at[pl.ds(cs, _B)]],
                         srcbufs.at[b], sems_s[b])
        pltpu.async_copy(emb_hbm.at[idxd_v.at[pl.ds(cs, _B)]],
                         dstbufs.at[b], sems_d[b])

    def drain(b):
        pltpu.make_async_copy(emb_hbm.at[idxs_v.at[pl.ds(0, _B)]],
                              srcbufs.at[b], sems_s[b]).wait()
        pltpu.make_async_copy(emb_hbm.at[idxd_v.at[pl.ds(0, _B)]],
                              dstbufs.at[b], sems_d[b]).wait()

    def compute(chunk, b):
        cs = chunk * _B

        def edge_body(e, carry2):
            acc = jnp.zeros((_L,), jnp.float32)
            for j in range(nj):
                a = srcbufs[b, e, pl.ds(j * _L, _L)]
                bb = dstbufs[b, e, pl.ds(j * _L, _L)]
                diff = a - bb
                acc = acc + diff * diff
            sums_v[pl.ds(e * _L, _L)] = plsc.cumsum(acc)
            return carry2

        lax.fori_loop(0, _B, edge_body, 0, unroll=2)

        def group_body(g, carry2):
            idx = g * (_L * _L) + lane15
            s = plsc.load_gather(sums_v, [idx])
            probs = 1.0 / (jnp.exp((s - _R) * (1.0 / _T)) + 1.0)
            outw_v[pl.ds(cs + g * _L, _L)] = probs
            return carry2

        lax.fori_loop(0, _B // _L, group_body, 0)

    for b in range(_NBUF):
        start(b, b)

    def ring_body(i, carry):
        for b in range(_NBUF):
            chunk = i * _NBUF + b
            drain(b)
            compute(chunk, b)
            nxt = chunk + _NBUF

            @pl.when(nxt < n_chunks)
            def _():
                start(nxt, b)
        return carry

    lax.fori_loop(0, n_chunks // _NBUF, ring_body, 0)
    pltpu.sync_copy(outw_v, out_hbm.at[pl.ds(base, ew)])


@jax.jit
def kernel(shower, embeddings, edge_index):
    del shower
    e_total = edge_index.shape[1]
    d = embeddings.shape[1]
    step = _B * _NBUF
    ew = (-(-e_total // _NW) + step - 1) // step * step
    n_chunks = ew // _B
    e_pad = ew * _NW

    src = edge_index[0]
    dst = edge_index[1]
    pad = e_pad - e_total
    if pad:
        zpad = jnp.zeros((pad,), jnp.int32)
        src = jnp.concatenate([src, zpad])
        dst = jnp.concatenate([dst, zpad])

    mesh = plsc.VectorSubcoreMesh(core_axis_name="c", subcore_axis_name="s")
    fn = pl.kernel(
        functools.partial(_sc_body, n_chunks),
        out_type=jax.ShapeDtypeStruct((e_pad,), jnp.float32),
        mesh=mesh,
        compiler_params=pltpu.CompilerParams(needs_layout_passes=False),
        scratch_types=[
            pltpu.VMEM((ew,), jnp.int32),
            pltpu.VMEM((ew,), jnp.int32),
            pltpu.VMEM((_NBUF, _B, d), jnp.float32),
            pltpu.VMEM((_NBUF, _B, d), jnp.float32),
            pltpu.VMEM((_B * _L,), jnp.float32),
            pltpu.VMEM((ew,), jnp.float32),
        ] + [pltpu.SemaphoreType.DMA] * (2 * _NBUF),
    )
    out = fn(embeddings, src, dst)
    return out[:e_total]


# speedup vs baseline: 1.2903x; 1.0710x over previous
import functools

import jax
import jax.numpy as jnp
from jax import lax
from jax.experimental import pallas as pl
from jax.experimental.pallas import tpu as pltpu
from jax.experimental.pallas import tpu_sc as plsc

_R = 2.0
_T = 1.0

_NC = 2
_NS = 16
_NW = _NC * _NS
_L = 16
_B = 64


def _sc_body(n_chunks, emb_hbm, idx_hbm, out_hbm,
             idx_v, rowbuf, sums_v, outw_v, sem):
    ew = n_chunks * _B
    nj = rowbuf.shape[1] // _L
    wid = lax.axis_index("s") * _NC + lax.axis_index("c")
    base = wid * ew

    pltpu.sync_copy(idx_hbm.at[pl.ds(2 * base, 2 * ew)], idx_v)

    lane15 = lax.iota(jnp.int32, _L) * _L + (_L - 1)

    def chunk_body(c, carry):
        cs = c * _B
        pltpu.async_copy(emb_hbm.at[idx_v.at[pl.ds(2 * cs, 2 * _B)]],
                         rowbuf, sem).wait()

        def edge_body(e, carry2):
            acc = jnp.zeros((_L,), jnp.float32)
            for j in range(nj):
                a = rowbuf[e, pl.ds(j * _L, _L)]
                bb = rowbuf[_B + e, pl.ds(j * _L, _L)]
                diff = a - bb
                acc = acc + diff * diff
            sums_v[pl.ds(e * _L, _L)] = plsc.cumsum(acc)
            return carry2

        lax.fori_loop(0, _B, edge_body, 0, unroll=2)

        def group_body(g, carry2):
            idx = g * (_L * _L) + lane15
            s = plsc.load_gather(sums_v, [idx])
            probs = 1.0 / (jnp.exp((s - _R) * (1.0 / _T)) + 1.0)
            outw_v[pl.ds(cs + g * _L, _L)] = probs
            return carry2

        lax.fori_loop(0, _B // _L, group_body, 0)
        return carry

    lax.fori_loop(0, n_chunks, chunk_body, 0)
    pltpu.sync_copy(outw_v, out_hbm.at[pl.ds(base, ew)])


@jax.jit
def kernel(shower, embeddings, edge_index):
    del shower
    e_total = edge_index.shape[1]
    d = embeddings.shape[1]
    ew = (-(-e_total // _NW) + _B - 1) // _B * _B
    n_chunks = ew // _B
    e_pad = ew * _NW

    src = edge_index[0]
    dst = edge_index[1]
    pad = e_pad - e_total
    if pad:
        zpad = jnp.zeros((pad,), jnp.int32)
        src = jnp.concatenate([src, zpad])
        dst = jnp.concatenate([dst, zpad])

    src_r = src.reshape(_NW * n_chunks, 1, _B)
    dst_r = dst.reshape(_NW * n_chunks, 1, _B)
    merged = jnp.concatenate([src_r, dst_r], axis=1).reshape(2 * e_pad)

    mesh = plsc.VectorSubcoreMesh(core_axis_name="c", subcore_axis_name="s")
    fn = pl.kernel(
        functools.partial(_sc_body, n_chunks),
        out_type=jax.ShapeDtypeStruct((e_pad,), jnp.float32),
        mesh=mesh,
        compiler_params=pltpu.CompilerParams(needs_layout_passes=False),
        scratch_types=[
            pltpu.VMEM((2 * ew,), jnp.int32),
            pltpu.VMEM((2 * _B, d), jnp.float32),
            pltpu.VMEM((_B * _L,), jnp.float32),
            pltpu.VMEM((ew,), jnp.float32),
            pltpu.SemaphoreType.DMA,
        ],
    )
    out = fn(embeddings, merged)
    return out[:e_total]
